# baseline (speedup 1.0000x reference)
import functools

import jax
import jax.numpy as jnp
from jax import lax
from jax.experimental import pallas as pl
from jax.experimental.pallas import tpu as pltpu
from jax.experimental.pallas import tpu_sc as plsc

BATCH = 16384
D_ID = 128
D_REVIEW = 64
CHUNK = 128


def _build_kernels():
    info = plsc.get_sparse_core_info()
    num_cores = info.num_cores
    num_workers = num_cores * info.num_subcores
    b_per_w = BATCH // num_workers
    n_chunks = b_per_w // CHUNK

    mesh = plsc.VectorSubcoreMesh(core_axis_name="c", subcore_axis_name="s")

    @functools.partial(
        pl.kernel,
        mesh=mesh,
        compiler_params=pltpu.CompilerParams(
            use_tc_tiling_on_sc=True, needs_layout_passes=False),
        out_type=[
            jax.ShapeDtypeStruct((BATCH, D_ID), jnp.float32),
            jax.ShapeDtypeStruct((BATCH, D_ID), jnp.float32),
        ],
        scratch_types=[
            pltpu.VMEM((n_chunks, CHUNK), jnp.int32),
            pltpu.VMEM((CHUNK, D_ID), jnp.float32),
            pltpu.VMEM((CHUNK, D_ID), jnp.float32),
            pltpu.VMEM((CHUNK, D_ID), jnp.float32),
            pltpu.VMEM((CHUNK, D_ID), jnp.float32),
            pltpu.SemaphoreType.DMA,
            pltpu.SemaphoreType.DMA,
            pltpu.SemaphoreType.DMA,
            pltpu.SemaphoreType.DMA,
        ],
    )
    def kernel_pe(idx_hbm, protos_hbm, emb_hbm, review_hbm,
                  proto_out, emb_out,
                  idx_v, pv0, ev0, pv1, ev1, gs0, gs1, os0, os1):
        del review_hbm
        wid = lax.axis_index("s") * num_cores + lax.axis_index("c")
        base = wid * b_per_w
        for j in range(n_chunks):
            pltpu.sync_copy(idx_hbm.at[pl.ds(base + j * CHUNK, CHUNK)],
                            idx_v.at[j])

        bufs = ((pv0, ev0, gs0, os0), (pv1, ev1, gs1, os1))

        def start_gather(j, s):
            pv, ev, gs, _ = bufs[s]
            return (
                pltpu.async_copy(protos_hbm.at[idx_v.at[j]], pv, gs),
                pltpu.async_copy(emb_hbm.at[idx_v.at[j]], ev, gs),
            )

        def start_copyout(j, s):
            pv, ev, _, os = bufs[s]
            off = base + j * CHUNK
            return (
                pltpu.async_copy(pv, proto_out.at[pl.ds(off, CHUNK)], os),
                pltpu.async_copy(ev, emb_out.at[pl.ds(off, CHUNK)], os),
            )

        gather_h = [None, None]
        copy_h = [None, None]
        for j in range(min(2, n_chunks)):
            gather_h[j] = start_gather(j, j)
        for j in range(n_chunks):
            s = j % 2
            if copy_h[s] is not None:
                for h in copy_h[s]:
                    h.wait()
                copy_h[s] = None
                gather_h[s] = start_gather(j, s)
            for h in gather_h[s]:
                h.wait()
            copy_h[s] = start_copyout(j, s)
        for s in range(2):
            if copy_h[s] is not None:
                for h in copy_h[s]:
                    h.wait()

    @functools.partial(
        pl.kernel,
        mesh=mesh,
        compiler_params=pltpu.CompilerParams(use_tc_tiling_on_sc=False),
        out_type=[
            jax.ShapeDtypeStruct((BATCH, D_REVIEW), jnp.float32),
        ],
        scratch_types=[
            pltpu.VMEM((n_chunks, CHUNK), jnp.int32),
            pltpu.VMEM((CHUNK, D_REVIEW), jnp.float32),
            pltpu.VMEM((CHUNK, D_REVIEW), jnp.float32),
            pltpu.SemaphoreType.DMA,
            pltpu.SemaphoreType.DMA,
            pltpu.SemaphoreType.DMA,
            pltpu.SemaphoreType.DMA,
        ],
    )
    def kernel_rev(idx_hbm, review_hbm, review_out,
                   idx_v, rv0, rv1, gs0, gs1, os0, os1):
        wid = lax.axis_index("s") * num_cores + lax.axis_index("c")
        base = wid * b_per_w
        for j in range(n_chunks):
            pltpu.sync_copy(idx_hbm.at[pl.ds(base + j * CHUNK, CHUNK)],
                            idx_v.at[j])

        bufs = ((rv0, gs0, os0), (rv1, gs1, os1))

        def start_gather(j, s):
            rv, gs, _ = bufs[s]
            return pltpu.async_copy(review_hbm.at[idx_v.at[j]], rv, gs)

        def start_copyout(j, s):
            rv, _, os = bufs[s]
            off = base + j * CHUNK
            return pltpu.async_copy(rv, review_out.at[pl.ds(off, CHUNK)], os)

        gather_h = [None, None]
        copy_h = [None, None]
        for j in range(min(2, n_chunks)):
            gather_h[j] = start_gather(j, j)
        for j in range(n_chunks):
            s = j % 2
            if copy_h[s] is not None:
                copy_h[s].wait()
                copy_h[s] = None
                gather_h[s] = start_gather(j, s)
            gather_h[s].wait()
            copy_h[s] = start_copyout(j, s)
        for s in range(2):
            if copy_h[s] is not None:
                copy_h[s].wait()

    return kernel_pe, kernel_rev


def kernel(nodes_u, global_protos, u_emb_weight, u_review_weight):
    kernel_pe, kernel_rev = _build_kernels()
    idx = nodes_u.astype(jnp.int32)
    proto_feats, u_id_feats = kernel_pe(
        idx, global_protos, u_emb_weight, u_review_weight)
    (u_review_feats,) = kernel_rev(idx, u_review_weight)
    return (proto_feats, u_id_feats, u_review_feats)


# speedup vs baseline: 1.2316x; 1.2316x over previous
import functools

import jax
import jax.numpy as jnp
from jax import lax
from jax.experimental import pallas as pl
from jax.experimental.pallas import tpu as pltpu
from jax.experimental.pallas import tpu_sc as plsc

BATCH = 16384
D_ID = 128
D_REVIEW = 64
CHUNK = 128


def _build_kernel():
    info = plsc.get_sparse_core_info()
    num_cores = info.num_cores
    num_workers = num_cores * info.num_subcores
    b_per_w = BATCH // num_workers
    n_chunks = b_per_w // CHUNK

    mesh = plsc.VectorSubcoreMesh(core_axis_name="c", subcore_axis_name="s")

    @functools.partial(
        pl.kernel,
        mesh=mesh,
        compiler_params=pltpu.CompilerParams(
            use_tc_tiling_on_sc=True, needs_layout_passes=False),
        out_type=[
            jax.ShapeDtypeStruct((BATCH, D_ID), jnp.float32),
            jax.ShapeDtypeStruct((BATCH, D_ID), jnp.float32),
            jax.ShapeDtypeStruct((BATCH, D_ID), jnp.float32),
        ],
        scratch_types=[
            pltpu.VMEM((n_chunks, CHUNK), jnp.int32),
            pltpu.VMEM((CHUNK, D_ID), jnp.float32),
            pltpu.VMEM((CHUNK, D_ID), jnp.float32),
            pltpu.VMEM((CHUNK, D_ID), jnp.float32),
            pltpu.VMEM((CHUNK, D_ID), jnp.float32),
            pltpu.VMEM((CHUNK, D_ID), jnp.float32),
            pltpu.VMEM((CHUNK, D_ID), jnp.float32),
            pltpu.SemaphoreType.DMA,
            pltpu.SemaphoreType.DMA,
            pltpu.SemaphoreType.DMA,
            pltpu.SemaphoreType.DMA,
        ],
    )
    def gather3(idx_hbm, protos_hbm, emb_hbm, review_hbm,
                proto_out, emb_out, review_out,
                idx_v, pv0, ev0, rv0, pv1, ev1, rv1,
                gs0, gs1, os0, os1):
        wid = lax.axis_index("s") * num_cores + lax.axis_index("c")
        base = wid * b_per_w
        for j in range(n_chunks):
            pltpu.sync_copy(idx_hbm.at[pl.ds(base + j * CHUNK, CHUNK)],
                            idx_v.at[j])

        bufs = ((pv0, ev0, rv0, gs0, os0), (pv1, ev1, rv1, gs1, os1))

        def start_gather(j, s):
            pv, ev, rv, gs, _ = bufs[s]
            return (
                pltpu.async_copy(protos_hbm.at[idx_v.at[j]], pv, gs),
                pltpu.async_copy(emb_hbm.at[idx_v.at[j]], ev, gs),
                pltpu.async_copy(review_hbm.at[idx_v.at[j]], rv, gs),
            )

        def start_copyout(j, s):
            pv, ev, rv, _, os = bufs[s]
            off = base + j * CHUNK
            return (
                pltpu.async_copy(pv, proto_out.at[pl.ds(off, CHUNK)], os),
                pltpu.async_copy(ev, emb_out.at[pl.ds(off, CHUNK)], os),
                pltpu.async_copy(rv, review_out.at[pl.ds(off, CHUNK)], os),
            )

        gather_h = [None, None]
        copy_h = [None, None]
        for j in range(min(2, n_chunks)):
            gather_h[j] = start_gather(j, j)
        for j in range(n_chunks):
            s = j % 2
            if copy_h[s] is not None:
                for h in copy_h[s]:
                    h.wait()
                copy_h[s] = None
                gather_h[s] = start_gather(j, s)
            for h in gather_h[s]:
                h.wait()
            copy_h[s] = start_copyout(j, s)
        for s in range(2):
            if copy_h[s] is not None:
                for h in copy_h[s]:
                    h.wait()

    return gather3


def kernel(nodes_u, global_protos, u_emb_weight, u_review_weight):
    gather3 = _build_kernel()
    idx = nodes_u.astype(jnp.int32)
    rev_pad = jnp.pad(u_review_weight, ((0, 0), (0, D_ID - D_REVIEW)))
    proto_feats, u_id_feats, review_wide = gather3(
        idx, global_protos, u_emb_weight, rev_pad)
    return (proto_feats, u_id_feats, review_wide[:, :D_REVIEW])
